# baseline (speedup 1.0000x reference)
import jax
import jax.numpy as jnp
from jax import lax
from jax.experimental import pallas as pl
from jax.experimental.pallas import tpu as pltpu
from jax.experimental.pallas import tpu_sc as plsc

_B = 16384
_F = 64
_L = 16
_NC = 2
_NS = 16
_NW = _NC * _NS
_PER_W = _B // _NW
_CHUNK = 128
_NCH = _PER_W // _CHUNK

_mesh = plsc.VectorSubcoreMesh(core_axis_name="c", subcore_axis_name="s")


def _sc_body(x_hbm, lab_hbm, cen_hbm, out_hbm, idx_v, x_v, rows_v, acc_v, sem):
    wid = lax.axis_index("s") * _NC + lax.axis_index("c")
    base = wid * _PER_W

    pltpu.sync_copy(lab_hbm.at[pl.ds(base, _PER_W)], idx_v)
    copies = [
        pltpu.async_copy(
            cen_hbm.at[idx_v.at[pl.ds(j * _CHUNK, _CHUNK)]],
            rows_v.at[pl.ds(j * _CHUNK, _CHUNK)],
            sem,
        )
        for j in range(_NCH)
    ]
    pltpu.sync_copy(x_hbm.at[pl.ds(base, _PER_W)], x_v)
    for c in copies:
        c.wait()

    def row_body(r, acc):
        for cc in range(_F // _L):
            xv = x_v[r, pl.ds(cc * _L, _L)]
            cv = rows_v[r, pl.ds(cc * _L, _L)]
            d = xv - cv
            acc = acc + d * d
        return acc

    acc = lax.fori_loop(0, _PER_W, row_body, jnp.zeros((_L,), jnp.float32))
    acc_v[...] = acc * (0.5 / _B)
    pltpu.sync_copy(acc_v, out_hbm.at[wid])


@jax.jit
def kernel(x, labels, centers):
    labels = labels.astype(jnp.int32)
    run = pl.kernel(
        _sc_body,
        out_type=jax.ShapeDtypeStruct((_NW, _L), jnp.float32),
        mesh=_mesh,
        compiler_params=pltpu.CompilerParams(use_tc_tiling_on_sc=False),
        scratch_types=[
            pltpu.VMEM((_PER_W,), jnp.int32),
            pltpu.VMEM((_PER_W, _F), jnp.float32),
            pltpu.VMEM((_PER_W, _F), jnp.float32),
            pltpu.VMEM((_L,), jnp.float32),
            pltpu.SemaphoreType.DMA,
        ],
    )
    partials = run(x, labels, centers)
    return jnp.sum(partials)
